# baseline (speedup 1.0000x reference)
import functools

import jax
import jax.numpy as jnp
from jax import lax
from jax.experimental import pallas as pl
from jax.experimental.pallas import tpu as pltpu
from jax.experimental.pallas import tpu_sc as plsc

N_FIELDS = 26
VOCAB = 100000
EMB_DIM = 32
B = 1024
L = 50

_info = plsc.get_sparse_core_info()
_NC, _NS, _LANES = _info.num_cores, _info.num_subcores, _info.num_lanes
_NW = _NC * _NS
_UNITS = N_FIELDS * L
_K = -(-_UNITS // _NW)
_JB = B // 128
_DT = EMB_DIM // 8


def _make_sc_gather():
    mesh = plsc.VectorSubcoreMesh(core_axis_name="c", subcore_axis_name="s")

    @functools.partial(
        pl.kernel,
        mesh=mesh,
        compiler_params=pltpu.CompilerParams(use_tc_tiling_on_sc=False, needs_layout_passes=False),
        out_type=jax.ShapeDtypeStruct((L, N_FIELDS, _DT, _JB, 8, 128),
                                      jnp.float32),
        scratch_types=[
            pltpu.VMEM((B,), jnp.int32),
            pltpu.VMEM((B, EMB_DIM), jnp.float32),
            pltpu.VMEM((_DT, _JB, 8, 128), jnp.float32),
            pltpu.SemaphoreType.DMA,
        ],
    )
    def k(tables_hbm, xc_hbm, out_hbm, idx_v, rows_v, stage_v, sem):
        wid = lax.axis_index("s") * _NC + lax.axis_index("c")
        iota16 = lax.iota(jnp.int32, _LANES)

        def unit_body(j, carry):
            u = wid + _NW * j

            @pl.when(u < _UNITS)
            def _do():
                f = u // L
                l = u % L
                pltpu.sync_copy(xc_hbm.at[f, l], idx_v)
                foff = f * VOCAB
                for s in range(B // _LANES):
                    idx_v[pl.ds(s * _LANES, _LANES)] = (
                        idx_v[pl.ds(s * _LANES, _LANES)] + foff)
                handles = [
                    pltpu.async_copy(
                        tables_hbm.at[idx_v.at[pl.ds(jb * 128, 128)]],
                        rows_v.at[pl.ds(jb * 128, 128)], sem)
                    for jb in range(_JB)
                ]
                for h in handles:
                    h.wait()

                def transpose_d(d, c2):
                    i = d // 8
                    r = d % 8
                    dsplat = jnp.full((_LANES,), d, jnp.int32)
                    for jb in range(_JB):
                        for cl in range(128 // _LANES):
                            bvec = jb * 128 + cl * _LANES + iota16
                            v = plsc.load_gather(rows_v, [bvec, dsplat])
                            stage_v[i, jb, r, pl.ds(cl * _LANES, _LANES)] = v
                    return c2

                lax.fori_loop(0, EMB_DIM, transpose_d, 0)
                pltpu.sync_copy(stage_v, out_hbm.at[l, f])

            return carry

        lax.fori_loop(0, _K, unit_body, 0)

    return k


_sc_gather = _make_sc_gather()


def kernel(x_cat, tables):
    flat_tables = tables.reshape(N_FIELDS * VOCAB, EMB_DIM)
    xc = jnp.transpose(x_cat, (2, 1, 0)).astype(jnp.int32)
    o6 = _sc_gather(flat_tables, xc)
    o = o6.transpose(0, 1, 2, 4, 3, 5).reshape(L, N_FIELDS, EMB_DIM, B)
    return o.transpose(3, 0, 1, 2)


# speedup vs baseline: 1.0380x; 1.0380x over previous
import functools

import jax
import jax.numpy as jnp
from jax import lax
from jax.experimental import pallas as pl
from jax.experimental.pallas import tpu as pltpu
from jax.experimental.pallas import tpu_sc as plsc

N_FIELDS = 26
VOCAB = 100000
EMB_DIM = 32
B = 1024
L = 50

_info = plsc.get_sparse_core_info()
_NC, _NS, _LANES = _info.num_cores, _info.num_subcores, _info.num_lanes
_NW = _NC * _NS
_UNITS = N_FIELDS * L
_K = -(-_UNITS // _NW)
_JB = B // 128
_DT = EMB_DIM // 8


def _make_sc_gather():
    mesh = plsc.VectorSubcoreMesh(core_axis_name="c", subcore_axis_name="s")

    @functools.partial(
        pl.kernel,
        mesh=mesh,
        compiler_params=pltpu.CompilerParams(
            use_tc_tiling_on_sc=False, needs_layout_passes=False),
        out_type=jax.ShapeDtypeStruct((L, N_FIELDS, _DT, _JB, 8, 128),
                                      jnp.float32),
        scratch_types=[
            pltpu.VMEM((B,), jnp.int32),
            pltpu.VMEM((B,), jnp.int32),
            pltpu.VMEM((B, EMB_DIM), jnp.float32),
            pltpu.VMEM((B, EMB_DIM), jnp.float32),
            pltpu.VMEM((_DT, _JB, 8, 128), jnp.float32),
            pltpu.SemaphoreType.DMA,
            pltpu.SemaphoreType.DMA,
        ],
    )
    def k(tables_hbm, xc_hbm, out_hbm,
          idx0_v, idx1_v, rows0_v, rows1_v, stage_v, sem0, sem1):
        wid = lax.axis_index("s") * _NC + lax.axis_index("c")
        iota16 = lax.iota(jnp.int32, _LANES)

        def unit_of(j):
            return wid + _NW * j

        def prefetch(u, idx_v, rows_v, sem):
            f = u // L
            l = u % L
            pltpu.sync_copy(xc_hbm.at[f, l], idx_v)
            foff = f * VOCAB
            for s in range(B // _LANES):
                idx_v[pl.ds(s * _LANES, _LANES)] = (
                    idx_v[pl.ds(s * _LANES, _LANES)] + foff)
            for jb in range(_JB):
                pltpu.async_copy(
                    tables_hbm.at[idx_v.at[pl.ds(jb * 128, 128)]],
                    rows_v.at[pl.ds(jb * 128, 128)], sem)

        def process(u, idx_v, rows_v, sem):
            f = u // L
            l = u % L
            for jb in range(_JB):
                pltpu.make_async_copy(
                    tables_hbm.at[idx_v.at[pl.ds(jb * 128, 128)]],
                    rows_v.at[pl.ds(jb * 128, 128)], sem).wait()

                def tloop(cl, carry, jb=jb):
                    bvec = jb * 128 + cl * _LANES + iota16
                    for d in range(EMB_DIM):
                        dsplat = jnp.full((_LANES,), d, jnp.int32)
                        v = plsc.load_gather(rows_v, [bvec, dsplat])
                        stage_v[d // 8, jb, d % 8,
                                pl.ds(cl * _LANES, _LANES)] = v
                    return carry

                lax.fori_loop(0, 128 // _LANES, tloop, 0)
            pltpu.sync_copy(stage_v, out_hbm.at[l, f])

        prefetch(unit_of(0), idx0_v, rows0_v, sem0)

        def body(m, carry):
            j0 = 2 * m
            j1 = 2 * m + 1
            j2 = 2 * m + 2

            @pl.when((j1 < _K) & (unit_of(j1) < _UNITS))
            def _p1():
                prefetch(unit_of(j1), idx1_v, rows1_v, sem1)

            @pl.when(unit_of(j0) < _UNITS)
            def _d0():
                process(unit_of(j0), idx0_v, rows0_v, sem0)

            @pl.when((j2 < _K) & (unit_of(j2) < _UNITS))
            def _p2():
                prefetch(unit_of(j2), idx0_v, rows0_v, sem0)

            @pl.when((j1 < _K) & (unit_of(j1) < _UNITS))
            def _d1():
                process(unit_of(j1), idx1_v, rows1_v, sem1)

            return carry

        lax.fori_loop(0, (_K + 1) // 2, body, 0)

    return k


_sc_gather = _make_sc_gather()


def kernel(x_cat, tables):
    flat_tables = tables.reshape(N_FIELDS * VOCAB, EMB_DIM)
    xc = jnp.transpose(x_cat, (2, 1, 0)).astype(jnp.int32)
    o6 = _sc_gather(flat_tables, xc)
    o = o6.transpose(0, 1, 2, 4, 3, 5).reshape(L, N_FIELDS, EMB_DIM, B)
    return o.transpose(3, 0, 1, 2)


# speedup vs baseline: 1.2614x; 1.2152x over previous
import functools

import jax
import jax.numpy as jnp
from jax import lax
from jax.experimental import pallas as pl
from jax.experimental.pallas import tpu as pltpu
from jax.experimental.pallas import tpu_sc as plsc

N_FIELDS = 26
VOCAB = 100000
EMB_DIM = 32
B = 1024
L = 50

_info = plsc.get_sparse_core_info()
_NC, _NS, _LANES = _info.num_cores, _info.num_subcores, _info.num_lanes
_NW = _NC * _NS
_UNITS = N_FIELDS * L
_K = -(-_UNITS // _NW)
_JB = B // 128
_DT = EMB_DIM // 8


def _make_sc_gather():
    mesh = plsc.VectorSubcoreMesh(core_axis_name="c", subcore_axis_name="s")

    @functools.partial(
        pl.kernel,
        mesh=mesh,
        compiler_params=pltpu.CompilerParams(
            use_tc_tiling_on_sc=False, needs_layout_passes=False),
        out_type=jax.ShapeDtypeStruct((L, N_FIELDS, _DT, _JB, 8, 128),
                                      jnp.float32),
        scratch_types=[
            pltpu.VMEM((B,), jnp.int32),
            pltpu.VMEM((B,), jnp.int32),
            pltpu.VMEM((B, EMB_DIM), jnp.float32),
            pltpu.VMEM((B, EMB_DIM), jnp.float32),
            pltpu.VMEM((_DT, _JB, 8, 128), jnp.float32),
            pltpu.SemaphoreType.DMA,
            pltpu.SemaphoreType.DMA,
        ],
    )
    def k(tables_hbm, xc_hbm, out_hbm,
          idx0_v, idx1_v, rows0_v, rows1_v, stage_v, sem0, sem1):
        wid = lax.axis_index("s") * _NC + lax.axis_index("c")
        iota16 = lax.iota(jnp.int32, _LANES)

        def unit_of(j):
            return wid + _NW * j

        def prefetch(u, idx_v, rows_v, sem):
            f = u // L
            l = u % L
            pltpu.sync_copy(xc_hbm.at[f, l], idx_v)
            foff = f * VOCAB
            for s in range(B // _LANES):
                idx_v[pl.ds(s * _LANES, _LANES)] = (
                    idx_v[pl.ds(s * _LANES, _LANES)] + foff)
            for jb in range(_JB):
                pltpu.async_copy(
                    tables_hbm.at[idx_v.at[pl.ds(jb * 128, 128)]],
                    rows_v.at[pl.ds(jb * 128, 128)], sem)

        def process(u, idx_v, rows_v, sem):
            f = u // L
            l = u % L
            for jb in range(_JB):
                pltpu.make_async_copy(
                    tables_hbm.at[idx_v.at[pl.ds(jb * 128, 128)]],
                    rows_v.at[pl.ds(jb * 128, 128)], sem).wait()

            @plsc.parallel_loop(0, B // _LANES, unroll=2)
            def _t(t):
                bvec = t * _LANES + iota16
                jb = t // 8
                cl = t % 8
                for d in range(EMB_DIM):
                    dsplat = jnp.full((_LANES,), d, jnp.int32)
                    v = plsc.load_gather(rows_v, [bvec, dsplat])
                    stage_v[d // 8, jb, d % 8,
                            pl.ds(cl * _LANES, _LANES)] = v

            pltpu.sync_copy(stage_v, out_hbm.at[l, f])

        prefetch(unit_of(0), idx0_v, rows0_v, sem0)

        def body(m, carry):
            j0 = 2 * m
            j1 = 2 * m + 1
            j2 = 2 * m + 2

            @pl.when((j1 < _K) & (unit_of(j1) < _UNITS))
            def _p1():
                prefetch(unit_of(j1), idx1_v, rows1_v, sem1)

            @pl.when(unit_of(j0) < _UNITS)
            def _d0():
                process(unit_of(j0), idx0_v, rows0_v, sem0)

            @pl.when((j2 < _K) & (unit_of(j2) < _UNITS))
            def _p2():
                prefetch(unit_of(j2), idx0_v, rows0_v, sem0)

            @pl.when((j1 < _K) & (unit_of(j1) < _UNITS))
            def _d1():
                process(unit_of(j1), idx1_v, rows1_v, sem1)

            return carry

        lax.fori_loop(0, (_K + 1) // 2, body, 0)

    return k


_sc_gather = _make_sc_gather()


def kernel(x_cat, tables):
    flat_tables = tables.reshape(N_FIELDS * VOCAB, EMB_DIM)
    xc = jnp.transpose(x_cat, (2, 1, 0)).astype(jnp.int32)
    o6 = _sc_gather(flat_tables, xc)
    o = o6.transpose(0, 1, 2, 4, 3, 5).reshape(L, N_FIELDS, EMB_DIM, B)
    return o.transpose(3, 0, 1, 2)
